# Initial kernel scaffold; baseline (speedup 1.0000x reference)
import jax
import jax.numpy as jnp
from jax.experimental import pallas as pl


def kernel(xs, ys, e_times, hs, h_times, neg_node, h_times_mask, emb_table, delta_table):
    raise NotImplementedError("write your pallas kernel here")


# speedup vs baseline: 1.1217x; 1.1217x over previous
import functools

import jax
import jax.numpy as jnp
from jax import lax
from jax.experimental import pallas as pl
from jax.experimental.pallas import tpu as pltpu
from jax.experimental.pallas import tpu_sc as plsc

B = 16384
H = 20
N = 5
D = 64

NC = 2
NS = 16
NW = NC * NS

CH = 512


def _sc_gather_body(emb_hbm, delta_hbm, xs_hbm, ys_hbm, hs_hbm, ns_hbm,
                    x_out, y_out, h_out, n_out, d_out,
                    idx_v, rows_v, dv, sem):
    wid = lax.axis_index("s") * NC + lax.axis_index("c")

    def run_job(idx_hbm, out_hbm, rows_per_worker):
        nch = rows_per_worker // CH
        base = wid * rows_per_worker

        def body(i, _):
            off = base + i * CH
            pltpu.sync_copy(idx_hbm.at[pl.ds(off, CH)], idx_v)
            pltpu.async_copy(emb_hbm.at[idx_v], rows_v, sem).wait()
            pltpu.sync_copy(rows_v, out_hbm.at[pl.ds(off, CH)])
            return 0

        lax.fori_loop(0, nch, body, 0)

    run_job(xs_hbm, x_out, B // NW)
    run_job(ys_hbm, y_out, B // NW)
    run_job(hs_hbm, h_out, B * H // NW)
    run_job(ns_hbm, n_out, B * N // NW)

    base = wid * (B // NW)
    pltpu.sync_copy(xs_hbm.at[pl.ds(base, B // NW)], idx_v.at[pl.ds(0, B // NW)])
    pltpu.async_copy(delta_hbm.at[idx_v.at[pl.ds(0, B // NW)]],
                     dv.at[pl.ds(0, B // NW)], sem).wait()
    pltpu.sync_copy(dv.at[pl.ds(0, B // NW)], d_out.at[pl.ds(base, B // NW)])


def _sc_gather(emb, delta_flat, xs, ys, hs_flat, ns_flat):
    mesh = plsc.VectorSubcoreMesh(core_axis_name="c", subcore_axis_name="s")
    f = pl.kernel(
        _sc_gather_body,
        out_type=[
            jax.ShapeDtypeStruct((B, D), jnp.float32),
            jax.ShapeDtypeStruct((B, D), jnp.float32),
            jax.ShapeDtypeStruct((B * H, D), jnp.float32),
            jax.ShapeDtypeStruct((B * N, D), jnp.float32),
            jax.ShapeDtypeStruct((B,), jnp.float32),
        ],
        mesh=mesh,
        compiler_params=pltpu.CompilerParams(use_tc_tiling_on_sc=False),
        scratch_types=[
            pltpu.VMEM((CH,), jnp.int32),
            pltpu.VMEM((CH, D), jnp.float32),
            pltpu.VMEM((CH,), jnp.float32),
            pltpu.SemaphoreType.DMA,
        ],
    )
    return f(emb, delta_flat, xs, ys, hs_flat, ns_flat)


def _log_sigmoid(z):
    return jnp.minimum(z, 0.0) - jnp.log1p(jnp.exp(-jnp.abs(z)))


def _tc_body(x_ref, y_ref, h_ref, n_ref, delta_ref, et_ref, ht_ref, mask_ref,
             loss_ref):
    x = x_ref[...]
    y = y_ref[...]
    h = h_ref[...]
    n = n_ref[...]
    delta = delta_ref[...]
    et = et_ref[...]
    ht = ht_ref[...]
    mask = mask_ref[...]

    d_time = jnp.abs(et - ht)
    dxy = x - y
    p_mu = -jnp.sum(dxy * dxy, axis=-1)

    dxh = x[:, None, :] - h
    alpha = -jnp.sum(dxh * dxh, axis=-1)
    amax = jnp.max(alpha, axis=1, keepdims=True)
    ea = jnp.exp(alpha - amax)
    attn = ea / jnp.sum(ea, axis=1, keepdims=True)
    decay = jnp.exp(delta * d_time) * mask
    w = attn * decay
    p_lambda = p_mu + jnp.sum(w * alpha, axis=-1)

    dxn = x[:, None, :] - n
    n_mu = -jnp.sum(dxn * dxn, axis=-1)

    hn2 = jnp.sum(h * h, axis=-1)
    W = jnp.sum(w, axis=1)
    S2 = jnp.sum(w * hn2, axis=1)
    hbar = jnp.sum(w[:, :, None] * h, axis=1)
    nn2 = jnp.sum(n * n, axis=-1)
    hdotn = jnp.sum(hbar[:, None, :] * n, axis=-1)
    n_lambda = (n_mu - S2[:, None] - W[:, None] * nn2 + 2.0 * hdotn)

    loss = _log_sigmoid(p_lambda) - jnp.sum(_log_sigmoid(n_lambda), axis=1)
    loss_ref[...] = loss[:, None]


def _tc_compute(x_e, y_e, h_e, n_e, delta, e_times, h_times, mask):
    BB = 512
    grid = (B // BB,)
    out = pl.pallas_call(
        _tc_body,
        grid=grid,
        in_specs=[
            pl.BlockSpec((BB, D), lambda i: (i, 0)),
            pl.BlockSpec((BB, D), lambda i: (i, 0)),
            pl.BlockSpec((BB, H, D), lambda i: (i, 0, 0)),
            pl.BlockSpec((BB, N, D), lambda i: (i, 0, 0)),
            pl.BlockSpec((BB, 1), lambda i: (i, 0)),
            pl.BlockSpec((BB, 1), lambda i: (i, 0)),
            pl.BlockSpec((BB, H), lambda i: (i, 0)),
            pl.BlockSpec((BB, H), lambda i: (i, 0)),
        ],
        out_specs=pl.BlockSpec((BB, 1), lambda i: (i, 0)),
        out_shape=jax.ShapeDtypeStruct((B, 1), jnp.float32),
    )(x_e, y_e, h_e, n_e, delta, e_times, h_times, mask)
    return out[:, 0]


def kernel(xs, ys, e_times, hs, h_times, neg_node, h_times_mask, emb_table,
           delta_table):
    xs = xs.astype(jnp.int32)
    ys = ys.astype(jnp.int32)
    hs_flat = hs.astype(jnp.int32).reshape(-1)
    ns_flat = neg_node.astype(jnp.int32).reshape(-1)
    delta_flat = delta_table.reshape(-1)

    x_e, y_e, h_e, n_e, delta = _sc_gather(emb_table, delta_flat, xs, ys,
                                           hs_flat, ns_flat)
    h_e = h_e.reshape(B, H, D)
    n_e = n_e.reshape(B, N, D)
    return _tc_compute(x_e, y_e, h_e, n_e, delta[:, None],
                       e_times[:, None], h_times, h_times_mask)
